# baseline (speedup 1.0000x reference)
import jax
import jax.numpy as jnp
from jax import lax
from jax.experimental import pallas as pl
from jax.experimental.pallas import tpu as pltpu

N, A, C = 64, 8732, 81
A4 = 4 * A
P = 2


def _tc_body(lp_ref, lt_ref, cp_ref, tt_ref, out_ref,
             mval_s, k_s, locrow_s, poscls_s):
    n = pl.program_id(0)
    ci = lax.broadcasted_iota(jnp.int32, (C, 1), 0)

    for j in range(P):
        r = n * P + j
        t = tt_ref[j]
        pos = t > 0

        g = jnp.sum(jnp.where(ci == t, cp_ref[j], 0.0), axis=0, keepdims=True)

        d = lp_ref[j] - lt_ref[j]
        ad = jnp.abs(d)
        sl1 = jnp.where(ad < 1.0, 0.5 * d * d, ad - 0.5)
        p4 = jnp.repeat(pos.astype(jnp.float32), 4, axis=1)
        loc_row = jnp.sum(sl1 * p4)

        pcnt = jnp.sum(pos.astype(jnp.int32))
        poscls_row = jnp.sum(jnp.where(pos, -g, 0.0))

        mval_s[pl.ds(r, 1), :] = jnp.where(pos, 0.0, g)
        k_s[pl.ds(r, 1), :] = (3 * pcnt)[None, None]
        locrow_s[pl.ds(r, 1), :] = loc_row[None, None]
        poscls_s[pl.ds(r, 1), :] = poscls_row[None, None]

    @pl.when(n == N // P - 1)
    def _():
        mval = mval_s[...]
        u = lax.bitcast_convert_type(mval, jnp.uint32)
        neg_sign = u >= jnp.uint32(0x80000000)
        key = jnp.where(neg_sign, ~u, u ^ jnp.uint32(0x80000000))
        k_raw = k_s[...]
        k_eff = jnp.minimum(k_raw, A)
        kr0 = jnp.maximum(k_eff, 1)

        def bit_step(i, carry):
            prefix, kr = carry
            b = (31 - i).astype(jnp.uint32)
            cond = (key >> b) == (prefix >> b)
            c = jnp.sum(cond.astype(jnp.int32), axis=1, keepdims=True)
            take1 = kr > c
            prefix = jnp.where(take1, prefix | (jnp.uint32(1) << b), prefix)
            kr = jnp.where(take1, kr - c, kr)
            return prefix, kr

        prefix, _ = lax.fori_loop(
            0, 32, bit_step, (jnp.zeros((N, 1), jnp.uint32), kr0))

        T = prefix
        less = key < T
        count_less = jnp.sum(less.astype(jnp.int32), axis=1, keepdims=True)
        sum_less = jnp.sum(jnp.where(less, mval, 0.0), axis=1, keepdims=True)
        neg_t = T < jnp.uint32(0x80000000)
        uT = jnp.where(neg_t, ~T, T ^ jnp.uint32(0x80000000))
        tval = lax.bitcast_convert_type(uT, jnp.float32)
        sel = sum_less + (k_eff - count_less).astype(jnp.float32) * tval
        sel = jnp.where(k_eff <= 0, 0.0, sel)

        cls_total = jnp.sum(poscls_s[...]) - jnp.sum(sel)
        num_pos = jnp.sum(k_raw).astype(jnp.float32) / 3.0
        loss = (jnp.sum(locrow_s[...]) + cls_total) / num_pos
        out_ref[...] = loss[None, None]


def kernel(loc_preds, loc_targets, cls_preds, cls_targets):
    t32 = cls_targets.astype(jnp.int32)
    tt = t32.reshape(N, 1, A)
    lpf = loc_preds.reshape(N, 1, A4)
    ltf = loc_targets.reshape(N, 1, A4)
    out = pl.pallas_call(
        _tc_body,
        grid=(N // P,),
        in_specs=[
            pl.BlockSpec((P, 1, A4), lambda n: (n, 0, 0)),
            pl.BlockSpec((P, 1, A4), lambda n: (n, 0, 0)),
            pl.BlockSpec((P, C, A), lambda n: (n, 0, 0)),
            pl.BlockSpec((P, 1, A), lambda n: (n, 0, 0)),
        ],
        out_specs=pl.BlockSpec((1, 1), lambda n: (0, 0)),
        out_shape=jax.ShapeDtypeStruct((1, 1), jnp.float32),
        scratch_shapes=[
            pltpu.VMEM((N, A), jnp.float32),
            pltpu.VMEM((N, 1), jnp.int32),
            pltpu.VMEM((N, 1), jnp.float32),
            pltpu.VMEM((N, 1), jnp.float32),
        ],
    )(lpf, ltf, cls_preds, tt)
    return out[0, 0]


# speedup vs baseline: 2.8410x; 2.8410x over previous
import jax
import jax.numpy as jnp
from jax import lax
from jax.experimental import pallas as pl
from jax.experimental.pallas import tpu as pltpu

N, A, C = 64, 8732, 81
A4 = 4 * A
P = 4


def _tc_body(lp_ref, lt_ref, p4_ref, cp_ref, tt_ref, out_ref,
             key_s, mval_s, k_s, locrow_s, poscls_s):
    n = pl.program_id(0)
    ci = lax.broadcasted_iota(jnp.int32, (C, 1), 0)

    for j in range(P):
        r = n * P + j
        t = tt_ref[j]
        pos = t > 0

        g = jnp.sum(jnp.where(ci == t, cp_ref[j], 0.0), axis=0, keepdims=True)

        d = lp_ref[j] - lt_ref[j]
        ad = jnp.abs(d)
        sl1 = jnp.where(ad < 1.0, 0.5 * d * d, ad - 0.5)
        loc_row = jnp.sum(sl1 * p4_ref[j])

        pcnt = jnp.sum(pos.astype(jnp.int32))
        poscls_row = jnp.sum(jnp.where(pos, -g, 0.0))

        masked = jnp.where(pos, 0.0, g)
        u = lax.bitcast_convert_type(masked, jnp.uint32)
        neg_sign = u >= jnp.uint32(0x80000000)
        key = jnp.where(neg_sign, ~u, u ^ jnp.uint32(0x80000000))

        key_s[pl.ds(r, 1), :] = key
        mval_s[pl.ds(r, 1), :] = masked
        k_s[pl.ds(r, 1), :] = (3 * pcnt)[None, None]
        locrow_s[pl.ds(r, 1), :] = loc_row[None, None]
        poscls_s[pl.ds(r, 1), :] = poscls_row[None, None]

    @pl.when(n == N // P - 1)
    def _():
        key = key_s[...]
        mval = mval_s[...]
        k_raw = k_s[...]
        k_eff = jnp.minimum(k_raw, A)
        kr0 = jnp.maximum(k_eff, 1)

        def bit_step(i, carry):
            prefix, kr = carry
            b = (31 - i).astype(jnp.uint32)
            cond = (key >> b) == (prefix >> b)
            c = jnp.sum(cond.astype(jnp.int32), axis=1, keepdims=True)
            take1 = kr > c
            prefix = jnp.where(take1, prefix | (jnp.uint32(1) << b), prefix)
            kr = jnp.where(take1, kr - c, kr)
            return prefix, kr

        prefix, _ = lax.fori_loop(
            0, 32, bit_step, (jnp.zeros((N, 1), jnp.uint32), kr0))

        T = prefix
        less = key < T
        count_less = jnp.sum(less.astype(jnp.int32), axis=1, keepdims=True)
        sum_less = jnp.sum(jnp.where(less, mval, 0.0), axis=1, keepdims=True)
        neg_t = T < jnp.uint32(0x80000000)
        uT = jnp.where(neg_t, ~T, T ^ jnp.uint32(0x80000000))
        tval = lax.bitcast_convert_type(uT, jnp.float32)
        sel = sum_less + (k_eff - count_less).astype(jnp.float32) * tval
        sel = jnp.where(k_eff <= 0, 0.0, sel)

        cls_total = jnp.sum(poscls_s[...]) - jnp.sum(sel)
        num_pos = jnp.sum(k_raw).astype(jnp.float32) / 3.0
        loss = (jnp.sum(locrow_s[...]) + cls_total) / num_pos
        out_ref[...] = loss[None, None]


def kernel(loc_preds, loc_targets, cls_preds, cls_targets):
    t32 = cls_targets.astype(jnp.int32)
    tt = t32.reshape(N, 1, A)
    lpf = loc_preds.reshape(N, 1, A4)
    ltf = loc_targets.reshape(N, 1, A4)
    posf4 = jnp.repeat((t32 > 0).astype(jnp.float32), 4, axis=1).reshape(N, 1, A4)
    out = pl.pallas_call(
        _tc_body,
        grid=(N // P,),
        in_specs=[
            pl.BlockSpec((P, 1, A4), lambda n: (n, 0, 0)),
            pl.BlockSpec((P, 1, A4), lambda n: (n, 0, 0)),
            pl.BlockSpec((P, 1, A4), lambda n: (n, 0, 0)),
            pl.BlockSpec((P, C, A), lambda n: (n, 0, 0)),
            pl.BlockSpec((P, 1, A), lambda n: (n, 0, 0)),
        ],
        out_specs=pl.BlockSpec((1, 1), lambda n: (0, 0)),
        out_shape=jax.ShapeDtypeStruct((1, 1), jnp.float32),
        scratch_shapes=[
            pltpu.VMEM((N, A), jnp.uint32),
            pltpu.VMEM((N, A), jnp.float32),
            pltpu.VMEM((N, 1), jnp.int32),
            pltpu.VMEM((N, 1), jnp.float32),
            pltpu.VMEM((N, 1), jnp.float32),
        ],
    )(lpf, ltf, posf4, cls_preds, tt)
    return out[0, 0]


# speedup vs baseline: 2.8420x; 1.0004x over previous
import jax
import jax.numpy as jnp
from jax import lax
from jax.experimental import pallas as pl
from jax.experimental.pallas import tpu as pltpu

N, A, C = 64, 8732, 81
A4 = 4 * A
P = 4


def _tc_body(lp_ref, lt_ref, p4_ref, cp_ref, tt_ref, out_ref,
             key_s, mval_s, k_s, locrow_s, poscls_s):
    n = pl.program_id(0)
    ci = lax.broadcasted_iota(jnp.int32, (C, 1), 0)

    for j in range(P):
        r = n * P + j
        t = tt_ref[j]
        pos = t > 0

        g = jnp.sum(jnp.where(ci == t, cp_ref[j], 0.0), axis=0, keepdims=True)

        d = lp_ref[j] - lt_ref[j]
        ad = jnp.abs(d)
        sl1 = jnp.where(ad < 1.0, 0.5 * d * d, ad - 0.5)
        loc_row = jnp.sum(sl1 * p4_ref[j])

        pcnt = jnp.sum(pos.astype(jnp.int32))
        poscls_row = jnp.sum(jnp.where(pos, -g, 0.0))

        masked = jnp.where(pos, 0.0, g)
        u = lax.bitcast_convert_type(masked, jnp.uint32)
        neg_sign = u >= jnp.uint32(0x80000000)
        key = jnp.where(neg_sign, ~u, u ^ jnp.uint32(0x80000000))

        key_s[pl.ds(r, 1), :] = key
        mval_s[pl.ds(r, 1), :] = masked
        k_s[pl.ds(r, 1), :] = (3 * pcnt)[None, None]
        locrow_s[pl.ds(r, 1), :] = loc_row[None, None]
        poscls_s[pl.ds(r, 1), :] = poscls_row[None, None]

    @pl.when(n == N // P - 1)
    def _():
        key = key_s[...]
        mval = mval_s[...]
        k_raw = k_s[...]
        k_eff = jnp.minimum(k_raw, A)
        kr0 = jnp.maximum(k_eff, 1)

        def bit_step(i, carry):
            prefix, kr = carry
            b = (31 - i).astype(jnp.uint32)
            cond = (key >> b) == (prefix >> b)
            c = jnp.sum(cond.astype(jnp.int32), axis=1, keepdims=True)
            take1 = kr > c
            prefix = jnp.where(take1, prefix | (jnp.uint32(1) << b), prefix)
            kr = jnp.where(take1, kr - c, kr)
            return prefix, kr

        prefix, _ = lax.fori_loop(
            0, 32, bit_step, (jnp.zeros((N, 1), jnp.uint32), kr0))

        T = prefix
        less = key < T
        count_less = jnp.sum(less.astype(jnp.int32), axis=1, keepdims=True)
        sum_less = jnp.sum(jnp.where(less, mval, 0.0), axis=1, keepdims=True)
        neg_t = T < jnp.uint32(0x80000000)
        uT = jnp.where(neg_t, ~T, T ^ jnp.uint32(0x80000000))
        tval = lax.bitcast_convert_type(uT, jnp.float32)
        sel = sum_less + (k_eff - count_less).astype(jnp.float32) * tval
        sel = jnp.where(k_eff <= 0, 0.0, sel)

        cls_total = jnp.sum(poscls_s[...]) - jnp.sum(sel)
        num_pos = jnp.sum(k_raw).astype(jnp.float32) / 3.0
        loss = (jnp.sum(locrow_s[...]) + cls_total) / num_pos
        out_ref[...] = loss[None, None]


def kernel(loc_preds, loc_targets, cls_preds, cls_targets):
    t32 = cls_targets.astype(jnp.int32)
    tt = t32.reshape(N, 1, A)
    lpf = loc_preds.reshape(N, 1, A4)
    ltf = loc_targets.reshape(N, 1, A4)
    posf4 = jnp.broadcast_to(
        (t32 > 0).astype(jnp.float32)[:, :, None], (N, A, 4)).reshape(N, 1, A4)
    out = pl.pallas_call(
        _tc_body,
        grid=(N // P,),
        in_specs=[
            pl.BlockSpec((P, 1, A4), lambda n: (n, 0, 0)),
            pl.BlockSpec((P, 1, A4), lambda n: (n, 0, 0)),
            pl.BlockSpec((P, 1, A4), lambda n: (n, 0, 0)),
            pl.BlockSpec((P, C, A), lambda n: (n, 0, 0)),
            pl.BlockSpec((P, 1, A), lambda n: (n, 0, 0)),
        ],
        out_specs=pl.BlockSpec((1, 1), lambda n: (0, 0)),
        out_shape=jax.ShapeDtypeStruct((1, 1), jnp.float32),
        scratch_shapes=[
            pltpu.VMEM((N, A), jnp.uint32),
            pltpu.VMEM((N, A), jnp.float32),
            pltpu.VMEM((N, 1), jnp.int32),
            pltpu.VMEM((N, 1), jnp.float32),
            pltpu.VMEM((N, 1), jnp.float32),
        ],
    )(lpf, ltf, posf4, cls_preds, tt)
    return out[0, 0]
